# baseline (speedup 1.0000x reference)
import functools

import jax
import jax.numpy as jnp
from jax import lax
from jax.experimental import pallas as pl
from jax.experimental.pallas import tpu as pltpu, tpu_sc as plsc

N_CLS = 1000
N_CTX = 4
CTX_DIM = 768
SUF_LEN = 72
SEQ_LEN = 1 + N_CTX + SUF_LEN
ROW = CTX_DIM
OUT_C = SEQ_LEN * ROW
SUF_C = SUF_LEN * ROW
CTX_SZ = N_CTX * ROW


def _sc_concat(prefix_hbm, ctx_hbm, suffix_hbm, out_hbm, ctx_v):
    info = plsc.get_sparse_core_info()
    nc = info.num_cores
    nw = nc * info.num_subcores
    wid = lax.axis_index("s") * nc + lax.axis_index("c")

    pltpu.sync_copy(ctx_hbm, ctx_v)

    per = N_CLS // nw
    rem = N_CLS - per * nw

    def copy_class(c):
        base = pl.multiple_of(c * OUT_C, 8)
        pltpu.sync_copy(prefix_hbm.at[pl.ds(pl.multiple_of(c * ROW, 8), ROW)],
                        out_hbm.at[pl.ds(base, ROW)])
        pltpu.sync_copy(ctx_v, out_hbm.at[pl.ds(base + ROW, CTX_SZ)])
        pltpu.sync_copy(suffix_hbm.at[pl.ds(pl.multiple_of(c * SUF_C, 8), SUF_C)],
                        out_hbm.at[pl.ds(base + ROW + CTX_SZ, SUF_C)])

    def body(i, carry):
        copy_class(wid + nw * i)
        return carry

    lax.fori_loop(0, per, body, None)

    @pl.when(wid < rem)
    def _tail():
        copy_class(per * nw + wid)


@jax.jit
def kernel(token_prefix, ctx, token_suffix):
    mesh = plsc.VectorSubcoreMesh(core_axis_name="c", subcore_axis_name="s")
    fn = functools.partial(
        pl.kernel,
        mesh=mesh,
        out_type=jax.ShapeDtypeStruct((N_CLS * OUT_C,), jnp.float32),
        scratch_types=[pltpu.VMEM((CTX_SZ,), jnp.float32)],
    )(_sc_concat)
    out = fn(token_prefix.reshape(-1), ctx.reshape(-1), token_suffix.reshape(-1))
    return out.reshape(N_CLS, SEQ_LEN, CTX_DIM)
